# baseline (speedup 1.0000x reference)
import jax
import jax.numpy as jnp
from jax.experimental import pallas as pl
from jax.experimental.pallas import tpu as pltpu

_SCORE_THRESH = 0.05
_NMS_THRESH = 0.5
_MAX_DET = 100
_N = 20000
_R = 160
_C = 128
_PAD = _R * _C
_NBLK = _R // 8
_NEG = -jnp.inf


def _nms_kernel(x1_ref, y1_ref, x2_ref, y2_ref, s_ref, out_ref, work_ref, area_ref):
    x1 = x1_ref[...]
    y1 = y1_ref[...]
    x2 = x2_ref[...]
    y2 = y2_ref[...]
    area_ref[...] = (x2 - x1) * (y2 - y1)
    s = s_ref[...]
    w0 = jnp.where(s > _SCORE_THRESH, s, _NEG)
    work_ref[...] = w0

    lane = jax.lax.broadcasted_iota(jnp.int32, (1, _C), 1)
    base_flat = (
        jax.lax.broadcasted_iota(jnp.int32, (8, _C), 0) * _C
        + jax.lax.broadcasted_iota(jnp.int32, (8, _C), 1)
    )

    def top3(neww):
        m1 = jnp.full((8, _C), _NEG, dtype=jnp.float32)
        m2 = m1
        m3 = m1
        f1 = jnp.full((8, _C), _PAD, dtype=jnp.int32)
        f2 = f1
        f3 = f1
        for v in range(_NBLK):
            xv = neww[v * 8:(v + 1) * 8, :]
            fx = base_flat + v * 8 * _C
            b1 = (xv > m1) | ((xv == m1) & (fx < f1))
            b2 = (xv > m2) | ((xv == m2) & (fx < f2))
            b3 = (xv > m3) | ((xv == m3) & (fx < f3))
            m3 = jnp.where(b2, m2, jnp.where(b3, xv, m3))
            f3 = jnp.where(b2, f2, jnp.where(b3, fx, f3))
            m2 = jnp.where(b1, m1, jnp.where(b2, xv, m2))
            f2 = jnp.where(b1, f1, jnp.where(b2, fx, f2))
            m1 = jnp.where(b1, xv, m1)
            f1 = jnp.where(b1, fx, f1)
        m1g = jnp.max(m1)
        f1g = jnp.min(jnp.where(m1 == m1g, f1, _PAD))
        win1 = f1 == f1g
        val2 = jnp.where(win1, m2, m1)
        fl2 = jnp.where(win1, f2, f1)
        m2g = jnp.max(val2)
        f2g = jnp.min(jnp.where(val2 == m2g, fl2, _PAD))
        win2 = fl2 == f2g
        val3 = jnp.where(win2, jnp.where(win1, m3, m2), val2)
        fl3 = jnp.where(win2, jnp.where(win1, f3, f2), fl2)
        m3g = jnp.max(val3)
        f3g = jnp.min(jnp.where(val3 == m3g, fl3, _PAD))
        return m1g, f1g, m2g, f2g, m3g, f3g

    def locate(flat):
        r = flat // _C
        sel = lane == flat % _C

        def pick(ref):
            return jnp.max(jnp.where(sel, ref[pl.ds(r, 1), :], _NEG))

        return pick(x1_ref), pick(y1_ref), pick(x2_ref), pick(y2_ref)

    pm1, pf1, pm2, pf2, pm3, pf3 = top3(w0)
    b_c = locate(pf1)
    s2_c = locate(pf2)

    def step(i, carry):
        (bm, bx1, by1, bx2, by2,
         s2m, s2f, s2x1, s2y1, s2x2, s2y2,
         s3m, s3f) = carry
        c3 = locate(s3f)
        valid = bm != _NEG
        row = (
            jnp.where(lane == 0, bx1, 0.0)
            + jnp.where(lane == 1, by1, 0.0)
            + jnp.where(lane == 2, bx2, 0.0)
            + jnp.where(lane == 3, by2, 0.0)
            + jnp.where(lane == 4, bm, 0.0)
        )
        out_ref[pl.ds(i, 1), :] = jnp.where(valid, row, 0.0)

        w = work_ref[...]
        xx1 = jnp.maximum(x1, bx1)
        yy1 = jnp.maximum(y1, by1)
        xx2 = jnp.minimum(x2, bx2)
        yy2 = jnp.minimum(y2, by2)
        inter = jnp.maximum(xx2 - xx1, 0.0) * jnp.maximum(yy2 - yy1, 0.0)
        barea = (bx2 - bx1) * (by2 - by1)
        iou = inter / (area_ref[...] + barea - inter + 1e-9)
        neww = jnp.where((iou > _NMS_THRESH) & valid, _NEG, w)
        work_ref[...] = neww
        m1, fl1, m2, fl2, m3, fl3 = top3(neww)

        def lookup(flat):
            return jax.lax.cond(
                flat == s2f,
                lambda: (s2x1, s2y1, s2x2, s2y2),
                lambda: jax.lax.cond(
                    flat == s3f,
                    lambda: c3,
                    lambda: locate(flat),
                ),
            )

        nb = lookup(fl1)
        ns2 = lookup(fl2)
        return (m1,) + nb + (m2, fl2) + ns2 + (m3, fl3)

    jax.lax.fori_loop(
        0,
        _MAX_DET,
        step,
        (pm1,) + b_c + (pm2, pf2) + s2_c + (pm3, pf3),
    )


def kernel(boxes, scores):
    pad_boxes = jnp.zeros((_PAD - _N, 4), dtype=boxes.dtype)
    b = jnp.concatenate([boxes, pad_boxes], axis=0)
    s = jnp.concatenate(
        [scores, jnp.full((_PAD - _N,), -1.0, dtype=scores.dtype)], axis=0
    ).reshape(_R, _C)
    x1 = b[:, 0].reshape(_R, _C)
    y1 = b[:, 1].reshape(_R, _C)
    x2 = b[:, 2].reshape(_R, _C)
    y2 = b[:, 3].reshape(_R, _C)
    out = pl.pallas_call(
        _nms_kernel,
        out_shape=jax.ShapeDtypeStruct((_MAX_DET, _C), jnp.float32),
        scratch_shapes=[
            pltpu.VMEM((_R, _C), jnp.float32),
            pltpu.VMEM((_R, _C), jnp.float32),
        ],
    )(x1, y1, x2, y2, s)
    return out[:, :5]


# speedup vs baseline: 1.7004x; 1.7004x over previous
import jax
import jax.numpy as jnp
from jax.experimental import pallas as pl
from jax.experimental.pallas import tpu as pltpu

_SCORE_THRESH = 0.05
_NMS_THRESH = 0.5
_MAX_DET = 100
_N = 20000
_R = 160
_C = 128
_PAD = _R * _C
_NBLK = _R // 8
_NEG = -jnp.inf


def _roll(x, k, axis):
    return pltpu.roll(x, k, axis)


def _nms_kernel(x1_ref, y1_ref, x2_ref, y2_ref, s_ref, out_ref, work_ref, area_ref):
    area_ref[...] = (x2_ref[...] - x1_ref[...]) * (y2_ref[...] - y1_ref[...])
    s = s_ref[...]
    work_ref[...] = jnp.where(s > _SCORE_THRESH, s, _NEG)

    lane = jax.lax.broadcasted_iota(jnp.int32, (1, _C), 1)
    row8 = jax.lax.broadcasted_iota(jnp.int32, (8, _C), 0)

    def step(i, carry):
        bval, bx1, by1, bx2, by2 = carry
        valid = bval != _NEG
        barea = (bx2 - bx1) * (by2 - by1)
        acc_v = jnp.full((8, _C), _NEG, dtype=jnp.float32)
        acc_r = jnp.zeros((8, _C), dtype=jnp.int32)
        for v in range(_NBLK):
            sl = pl.ds(v * 8, 8)
            xv1 = x1_ref[sl, :]
            yv1 = y1_ref[sl, :]
            xv2 = x2_ref[sl, :]
            yv2 = y2_ref[sl, :]
            wv = work_ref[sl, :]
            xx1 = jnp.maximum(xv1, bx1)
            yy1 = jnp.maximum(yv1, by1)
            xx2 = jnp.minimum(xv2, bx2)
            yy2 = jnp.minimum(yv2, by2)
            inter = jnp.maximum(xx2 - xx1, 0.0) * jnp.maximum(yy2 - yy1, 0.0)
            iou = inter / (area_ref[sl, :] + barea - inter + 1e-9)
            nw = jnp.where((iou > _NMS_THRESH) & valid, _NEG, wv)
            work_ref[sl, :] = nw
            gt = nw > acc_v
            acc_r = jnp.where(gt, row8 + v * 8, acc_r)
            acc_v = jnp.where(gt, nw, acc_v)
        for k in (4, 2, 1):
            rv = _roll(acc_v, k, 0)
            rr = _roll(acc_r, k, 0)
            b = (rv > acc_v) | ((rv == acc_v) & (rr < acc_r))
            acc_v = jnp.where(b, rv, acc_v)
            acc_r = jnp.where(b, rr, acc_r)
        colv = acc_v[0:1, :]
        colr = acc_r[0:1, :]
        g1 = jnp.zeros((8, _C), dtype=jnp.float32)
        g2 = g1
        g3 = g1
        g4 = g1
        for v in range(_NBLK):
            sl = pl.ds(v * 8, 8)
            msk = (row8 + v * 8) == colr
            g1 = g1 + jnp.where(msk, x1_ref[sl, :], 0.0)
            g2 = g2 + jnp.where(msk, y1_ref[sl, :], 0.0)
            g3 = g3 + jnp.where(msk, x2_ref[sl, :], 0.0)
            g4 = g4 + jnp.where(msk, y2_ref[sl, :], 0.0)
        for k in (4, 2, 1):
            g1 = g1 + _roll(g1, k, 0)
            g2 = g2 + _roll(g2, k, 0)
            g3 = g3 + _roll(g3, k, 0)
            g4 = g4 + _roll(g4, k, 0)
        c1 = g1[0:1, :]
        c2 = g2[0:1, :]
        c3 = g3[0:1, :]
        c4 = g4[0:1, :]
        tv = colv
        tf = colr * _C + lane
        for k in (64, 32, 16, 8, 4, 2, 1):
            rv = _roll(tv, k, 1)
            rf = _roll(tf, k, 1)
            r1 = _roll(c1, k, 1)
            r2 = _roll(c2, k, 1)
            r3 = _roll(c3, k, 1)
            r4 = _roll(c4, k, 1)
            b = (rv > tv) | ((rv == tv) & (rf < tf))
            tv = jnp.where(b, rv, tv)
            tf = jnp.where(b, rf, tf)
            c1 = jnp.where(b, r1, c1)
            c2 = jnp.where(b, r2, c2)
            c3 = jnp.where(b, r3, c3)
            c4 = jnp.where(b, r4, c4)
        nvalid = tv != _NEG
        rowv = (
            jnp.where(lane == 0, c1, 0.0)
            + jnp.where(lane == 1, c2, 0.0)
            + jnp.where(lane == 2, c3, 0.0)
            + jnp.where(lane == 3, c4, 0.0)
            + jnp.where(lane == 4, tv, 0.0)
        )
        out_ref[pl.ds(i, 1), :] = jnp.where(nvalid, rowv, 0.0)
        return (tv, c1, c2, c3, c4)

    zero = jnp.zeros((1, _C), dtype=jnp.float32)
    init = (jnp.full((1, _C), _NEG, dtype=jnp.float32), zero, zero, zero, zero)
    jax.lax.fori_loop(0, _MAX_DET, step, init)


def kernel(boxes, scores):
    pad_boxes = jnp.zeros((_PAD - _N, 4), dtype=boxes.dtype)
    b = jnp.concatenate([boxes, pad_boxes], axis=0)
    s = jnp.concatenate(
        [scores, jnp.full((_PAD - _N,), -1.0, dtype=scores.dtype)], axis=0
    ).reshape(_R, _C)
    x1 = b[:, 0].reshape(_R, _C)
    y1 = b[:, 1].reshape(_R, _C)
    x2 = b[:, 2].reshape(_R, _C)
    y2 = b[:, 3].reshape(_R, _C)
    out = pl.pallas_call(
        _nms_kernel,
        out_shape=jax.ShapeDtypeStruct((_MAX_DET, _C), jnp.float32),
        scratch_shapes=[
            pltpu.VMEM((_R, _C), jnp.float32),
            pltpu.VMEM((_R, _C), jnp.float32),
        ],
    )(x1, y1, x2, y2, s)
    return out[:, :5]


# speedup vs baseline: 2.4744x; 1.4552x over previous
import jax
import jax.numpy as jnp
from jax.experimental import pallas as pl
from jax.experimental.pallas import tpu as pltpu

_SCORE_THRESH = 0.05
_NMS_THRESH = 0.5
_MAX_DET = 100
_N = 20000
_R = 160
_C = 128
_PAD = _R * _C
_NBLK = _R // 8
_NEG = -jnp.inf


def _nms_kernel(x1_ref, y1_ref, x2_ref, y2_ref, s_ref, out_ref, work_ref, area_ref):
    area_ref[...] = (x2_ref[...] - x1_ref[...]) * (y2_ref[...] - y1_ref[...])
    s = s_ref[...]
    work_ref[...] = jnp.where(s > _SCORE_THRESH, s, _NEG)

    lane = jax.lax.broadcasted_iota(jnp.int32, (1, _C), 1)
    row8 = jax.lax.broadcasted_iota(jnp.int32, (8, _C), 0)

    def step(i, carry):
        bval, bx1, by1, bx2, by2 = carry
        valid = bval != _NEG
        barea = (bx2 - bx1) * (by2 - by1)
        acc_v = jnp.full((8, _C), _NEG, dtype=jnp.float32)
        acc_r = jnp.zeros((8, _C), dtype=jnp.int32)
        for v in range(_NBLK):
            sl = pl.ds(v * 8, 8)
            xv1 = x1_ref[sl, :]
            yv1 = y1_ref[sl, :]
            xv2 = x2_ref[sl, :]
            yv2 = y2_ref[sl, :]
            wv = work_ref[sl, :]
            xx1 = jnp.maximum(xv1, bx1)
            yy1 = jnp.maximum(yv1, by1)
            xx2 = jnp.minimum(xv2, bx2)
            yy2 = jnp.minimum(yv2, by2)
            inter = jnp.maximum(xx2 - xx1, 0.0) * jnp.maximum(yy2 - yy1, 0.0)
            iou = inter / (area_ref[sl, :] + barea - inter + 1e-9)
            nw = jnp.where((iou > _NMS_THRESH) & valid, _NEG, wv)
            work_ref[sl, :] = nw
            gt = nw > acc_v
            acc_r = jnp.where(gt, row8 + v * 8, acc_r)
            acc_v = jnp.where(gt, nw, acc_v)
        for k in (4, 2, 1):
            rv = pltpu.roll(acc_v, k, 0)
            rr = pltpu.roll(acc_r, k, 0)
            b = (rv > acc_v) | ((rv == acc_v) & (rr < acc_r))
            acc_v = jnp.where(b, rv, acc_v)
            acc_r = jnp.where(b, rr, acc_r)
        colv = acc_v[0:1, :]
        colr = acc_r[0:1, :]
        g1 = jnp.zeros((8, _C), dtype=jnp.float32)
        g2 = g1
        g3 = g1
        g4 = g1
        for v in range(_NBLK):
            sl = pl.ds(v * 8, 8)
            msk = (row8 + v * 8) == colr
            g1 = g1 + jnp.where(msk, x1_ref[sl, :], 0.0)
            g2 = g2 + jnp.where(msk, y1_ref[sl, :], 0.0)
            g3 = g3 + jnp.where(msk, x2_ref[sl, :], 0.0)
            g4 = g4 + jnp.where(msk, y2_ref[sl, :], 0.0)
        for k in (4, 2, 1):
            g1 = g1 + pltpu.roll(g1, k, 0)
            g2 = g2 + pltpu.roll(g2, k, 0)
            g3 = g3 + pltpu.roll(g3, k, 0)
            g4 = g4 + pltpu.roll(g4, k, 0)
        c1 = g1[0:1, :]
        c2 = g2[0:1, :]
        c3 = g3[0:1, :]
        c4 = g4[0:1, :]
        m1 = jnp.max(colv, axis=1, keepdims=True)
        sel = colv == m1
        flatf = (colr * _C + lane).astype(jnp.float32)
        pk = jnp.concatenate(
            [
                jnp.where(sel, -flatf, _NEG),
                jnp.where(sel, c1, _NEG),
                jnp.where(sel, c2, _NEG),
                jnp.where(sel, c3, _NEG),
                jnp.where(sel, c4, _NEG),
                jnp.where(sel, flatf, _NEG),
                jnp.where(sel, flatf, _NEG),
                jnp.where(sel, flatf, _NEG),
            ],
            axis=0,
        )
        r8 = jnp.max(pk, axis=1, keepdims=True)
        negf = r8[0:1, :]
        fx1 = r8[1:2, :]
        fy1 = r8[2:3, :]
        fx2 = r8[3:4, :]
        fy2 = r8[4:5, :]
        posf = r8[5:6, :]
        tie_s = (posf + negf)[0, 0] != 0.0

        def fix():
            um = flatf == -negf
            pk2 = jnp.concatenate(
                [
                    jnp.where(um, c1, _NEG),
                    jnp.where(um, c2, _NEG),
                    jnp.where(um, c3, _NEG),
                    jnp.where(um, c4, _NEG),
                ]
                + [jnp.where(um, c4, _NEG)] * 4,
                axis=0,
            )
            q8 = jnp.max(pk2, axis=1, keepdims=True)
            return q8[0:1, :], q8[1:2, :], q8[2:3, :], q8[3:4, :]

        nx1, ny1, nx2, ny2 = jax.lax.cond(
            tie_s, fix, lambda: (fx1, fy1, fx2, fy2)
        )
        nvalid = m1 != _NEG
        rowv = (
            jnp.where(lane == 0, nx1, 0.0)
            + jnp.where(lane == 1, ny1, 0.0)
            + jnp.where(lane == 2, nx2, 0.0)
            + jnp.where(lane == 3, ny2, 0.0)
            + jnp.where(lane == 4, m1, 0.0)
        )
        out_ref[pl.ds(i, 1), :] = jnp.where(nvalid, rowv, 0.0)
        return (m1, nx1, ny1, nx2, ny2)

    z = jnp.zeros((1, 1), dtype=jnp.float32)
    init = (jnp.full((1, 1), _NEG, dtype=jnp.float32), z, z, z, z)
    jax.lax.fori_loop(0, _MAX_DET, step, init)


def kernel(boxes, scores):
    pad_boxes = jnp.zeros((_PAD - _N, 4), dtype=boxes.dtype)
    b = jnp.concatenate([boxes, pad_boxes], axis=0)
    s = jnp.concatenate(
        [scores, jnp.full((_PAD - _N,), -1.0, dtype=scores.dtype)], axis=0
    ).reshape(_R, _C)
    x1 = b[:, 0].reshape(_R, _C)
    y1 = b[:, 1].reshape(_R, _C)
    x2 = b[:, 2].reshape(_R, _C)
    y2 = b[:, 3].reshape(_R, _C)
    out = pl.pallas_call(
        _nms_kernel,
        out_shape=jax.ShapeDtypeStruct((_MAX_DET, _C), jnp.float32),
        scratch_shapes=[
            pltpu.VMEM((_R, _C), jnp.float32),
            pltpu.VMEM((_R, _C), jnp.float32),
        ],
    )(x1, y1, x2, y2, s)
    return out[:, :5]
